# baseline (speedup 1.0000x reference)
import functools

import jax
import jax.numpy as jnp
from jax import lax
from jax.experimental import pallas as pl
from jax.experimental.pallas import tpu as pltpu
from jax.experimental.pallas import tpu_sc as plsc

N_SRC = 100000
N_DST = 12288
T = 4
HID = 128
E = 1600000

L = 16
NC = 2
NS = 16
CHC = 512
NB = CHC // 128
NCHUNK = E // CHC
NPAIR = 97
XW = 6256
N_SRC_P = NS * XW
DPT = N_DST // NS


def _rsqrt_newton(c):
    y = plsc.bitcast(jnp.int32(0x5F3759DF) - (plsc.bitcast(c, jnp.int32) >> 1),
                     jnp.float32)
    for _ in range(3):
        y = y * (jnp.float32(1.5) - jnp.float32(0.5) * c * y * y)
    return y


_sc_mesh = plsc.VectorSubcoreMesh(core_axis_name="c", subcore_axis_name="s")


@functools.partial(
    pl.kernel,
    out_type=jax.ShapeDtypeStruct((T, N_DST), jnp.float32),
    mesh=_sc_mesh,
    compiler_params=pltpu.CompilerParams(needs_layout_passes=False),
    scratch_types=[
        pltpu.VMEM_SHARED((N_SRC_P,), jnp.float32),
        pltpu.VMEM_SHARED((N_SRC_P,), jnp.float32),
        pltpu.VMEM_SHARED((N_SRC_P,), jnp.float32),
        pltpu.VMEM_SHARED((N_SRC_P,), jnp.float32),
        pltpu.VMEM_SHARED((NS, N_DST), jnp.float32),
        pltpu.VMEM((N_DST,), jnp.float32),
        pltpu.VMEM((N_DST,), jnp.float32),
        pltpu.VMEM((N_DST,), jnp.float32),
        pltpu.VMEM((N_DST,), jnp.float32),
        pltpu.VMEM((T, CHC), jnp.int32),
        pltpu.VMEM((T, CHC), jnp.int32),
        pltpu.VMEM((T, CHC), jnp.int32),
        pltpu.VMEM((T, CHC), jnp.int32),
        pltpu.VMEM((2, NB, 128), jnp.float32),
        pltpu.VMEM((2, NB, 128), jnp.float32),
        pltpu.VMEM((128,), jnp.float32),
        pltpu.VMEM((2048,), jnp.float32),
        pltpu.VMEM((XW,), jnp.float32),
        pltpu.VMEM((XW,), jnp.float32),
        pltpu.VMEM((DPT,), jnp.float32),
        pltpu.VMEM((DPT,), jnp.float32),
        pltpu.VMEM((DPT,), jnp.float32),
        pltpu.VMEM((DPT,), jnp.float32),
        pltpu.SemaphoreType.DMA,
        pltpu.SemaphoreType.DMA,
        pltpu.SemaphoreType.DMA,
        pltpu.SemaphoreType.DMA,
    ],
)
def _sc_graph_agg(esrc, edst, xpad, out,
                  dg0_sp, dg1_sp, val0_sp, val1_sp, red_sp,
                  agg0_v, agg1_v, din0_v, din1_v,
                  sA, sB, dA, dB, gA, gB, ones_v, zbuf,
                  xbuf, cbuf, abuf, ibuf, c1, obuf,
                  semLA, semLB, semA, semB):
    c = lax.axis_index("c")
    s = lax.axis_index("s")
    base_chunk = 195 * s + jnp.minimum(s, 5)
    nchunks = jnp.where(s < 5, 196, 195)
    dg_sps = (dg0_sp, dg1_sp)
    val_sps = (val0_sp, val1_sp)
    agg_vs = (agg0_v, agg1_v)
    din_vs = (din0_v, din1_v)

    def _init(i, _):
        zbuf[pl.ds(i * L, L)] = jnp.zeros((L,), jnp.float32)
        return 0
    lax.fori_loop(0, 2048 // L, _init, 0)
    for i in range(128 // L):
        ones_v[pl.ds(i * L, L)] = jnp.ones((L,), jnp.float32)
    ones16 = jnp.ones((L,), jnp.float32)

    def col0_of(ck):
        return (base_chunk + ck) * CHC

    def lin(ref, ck, buf, sem):
        return pltpu.async_copy(ref.at[:, pl.ds(col0_of(ck), CHC)], buf, sem)

    def lin_wait(ref, ck, buf, sem):
        pltpu.make_async_copy(ref.at[:, pl.ds(col0_of(ck), CHC)],
                              buf, sem).wait()

    for tl in range(2):
        for q in range(3):
            pltpu.sync_copy(zbuf,
                            dg_sps[tl].at[pl.ds(s * XW + q * 2048, 2048)])
        pltpu.sync_copy(zbuf.at[pl.ds(0, XW - 3 * 2048)],
                        dg_sps[tl].at[pl.ds(s * XW + 3 * 2048,
                                            XW - 3 * 2048)])

    def _zero(i, _):
        z = jnp.zeros((L,), jnp.float32)
        agg0_v[pl.ds(i * L, L)] = z
        agg1_v[pl.ds(i * L, L)] = z
        din0_v[pl.ds(i * L, L)] = z
        din1_v[pl.ds(i * L, L)] = z
        return 0
    lax.fori_loop(0, N_DST // L, _zero, 0)
    plsc.subcore_barrier()

    def scat_chunk(sbuf, sem):
        cps = []
        for tl in range(2):
            r = c * 2 + tl
            for j in range(NB):
                cps.append(pltpu.async_copy(
                    ones_v,
                    dg_sps[tl].at[sbuf.at[r, pl.ds(j * 128, 128)]],
                    sem, add=True))
        return cps

    def drain(cps):
        for cp in cps:
            cp.wait()

    lin(esrc, 0, sA, semLA)

    def _p1(p, _):
        c0 = 2 * p
        lin_wait(esrc, c0, sA, semLA)
        cpsA = scat_chunk(sA, semA)
        lin(esrc, c0 + 1, sB, semLB)
        drain(cpsA)
        lin_wait(esrc, c0 + 1, sB, semLB)
        cpsB = scat_chunk(sB, semB)

        @pl.when(c0 + 2 < nchunks)
        def _():
            lin(esrc, c0 + 2, sA, semLA)
        drain(cpsB)
        return 0
    lax.fori_loop(0, NPAIR, _p1, 0)

    @pl.when(s < 5)
    def _():
        lin(esrc, 195, sB, semLB)
    lin_wait(esrc, 194, sA, semLA)
    drain(scat_chunk(sA, semA))

    @pl.when(s < 5)
    def _():
        lin_wait(esrc, 195, sB, semLB)
        drain(scat_chunk(sB, semB))

    plsc.subcore_barrier()

    for tl in range(2):
        t = c * 2 + tl
        pltpu.sync_copy(xpad.at[t, s, :], xbuf)
        pltpu.sync_copy(dg_sps[tl].at[pl.ds(s * XW, XW)], cbuf)

        def _val(i, _):
            xv = xbuf[pl.ds(i * L, L)]
            xv = jnp.where(xv == xv, xv, jnp.float32(0.0))
            cv = jnp.maximum(cbuf[pl.ds(i * L, L)], jnp.float32(1.0))
            xbuf[pl.ds(i * L, L)] = xv * _rsqrt_newton(cv)
            return 0
        lax.fori_loop(0, XW // L, _val, 0)
        pltpu.sync_copy(xbuf, val_sps[tl].at[pl.ds(s * XW, XW)])
    plsc.subcore_barrier()

    def gath_chunk(sbuf, gbuf, sem):
        cps = []
        for tl in range(2):
            r = c * 2 + tl
            for j in range(NB):
                cps.append(pltpu.async_copy(
                    val_sps[tl].at[sbuf.at[r, pl.ds(j * 128, 128)]],
                    gbuf.at[tl, j], sem))
        return cps

    def consume(dbuf, gbuf):
        for tl in range(2):
            r = c * 2 + tl

            def _row(j, _):
                for i in range(128 // L):
                    dv = dbuf[r, pl.ds(j * 128 + i * L, L)]
                    gv = gbuf[tl, j, pl.ds(i * L, L)]
                    plsc.addupdate_scatter(agg_vs[tl], [dv], gv)
                    plsc.addupdate_scatter(din_vs[tl], [dv], ones16)
                return 0
            lax.fori_loop(0, NB, _row, 0)

    lin(esrc, 0, sA, semLA)
    lin(edst, 0, dA, semLA)

    def _p2(p, _):
        c0 = 2 * p
        lin_wait(esrc, c0, sA, semLA)
        lin_wait(edst, c0, dA, semLA)
        cpsA = gath_chunk(sA, gA, semA)
        lin(esrc, c0 + 1, sB, semLB)
        lin(edst, c0 + 1, dB, semLB)
        drain(cpsA)
        lin_wait(esrc, c0 + 1, sB, semLB)
        lin_wait(edst, c0 + 1, dB, semLB)
        cpsB = gath_chunk(sB, gB, semB)
        consume(dA, gA)

        @pl.when(c0 + 2 < nchunks)
        def _():
            lin(esrc, c0 + 2, sA, semLA)
            lin(edst, c0 + 2, dA, semLA)
        drain(cpsB)
        consume(dB, gB)
        return 0
    lax.fori_loop(0, NPAIR, _p2, 0)

    @pl.when(s < 5)
    def _():
        lin(esrc, 195, sB, semLB)
        lin(edst, 195, dB, semLB)
    lin_wait(esrc, 194, sA, semLA)
    lin_wait(edst, 194, dA, semLA)
    drain(gath_chunk(sA, gA, semA))
    consume(dA, gA)

    @pl.when(s < 5)
    def _():
        lin_wait(esrc, 195, sB, semLB)
        lin_wait(edst, 195, dB, semLB)
        drain(gath_chunk(sB, gB, semB))
        consume(dB, gB)

    for tl in range(2):
        t = c * 2 + tl
        for which, acc in ((0, abuf), (1, ibuf)):
            src_v = agg_vs[tl] if which == 0 else din_vs[tl]
            plsc.subcore_barrier()
            pltpu.sync_copy(src_v, red_sp.at[s])
            plsc.subcore_barrier()
            pltpu.sync_copy(red_sp.at[0, pl.ds(s * DPT, DPT)], acc)
            for r in range(1, NS):
                pltpu.sync_copy(red_sp.at[r, pl.ds(s * DPT, DPT)], c1)

                def _acc(i, _):
                    acc[pl.ds(i * L, L)] = (acc[pl.ds(i * L, L)]
                                            + c1[pl.ds(i * L, L)])
                    return 0
                lax.fori_loop(0, DPT // L, _acc, 0)

        def _scale(i, _):
            a = abuf[pl.ds(i * L, L)]
            d = jnp.maximum(ibuf[pl.ds(i * L, L)], jnp.float32(1.0))
            obuf[pl.ds(i * L, L)] = a * _rsqrt_newton(d)
            return 0
        lax.fori_loop(0, DPT // L, _scale, 0)
        pltpu.sync_copy(obuf, out.at[t, pl.ds(s * DPT, DPT)])


def _tc_expand_body(agg_ref, w_ref, b_ref, out_ref):
    for t in range(T):
        a = agg_ref[t, :]
        y = a[:, None] * w_ref[t, 0, :][None, :] + b_ref[t, :][None, :]
        out_ref[:, 0, t, :] = jnp.where(y > 0, y, jnp.float32(0.01) * y)


def _tc_expand(aggs, W, b):
    BN = 1024
    grid = (N_DST // BN,)
    return pl.pallas_call(
        _tc_expand_body,
        grid=grid,
        in_specs=[
            pl.BlockSpec((T, BN), lambda i: (0, i)),
            pl.BlockSpec((T, 1, HID), lambda i: (0, 0, 0)),
            pl.BlockSpec((T, HID), lambda i: (0, 0)),
        ],
        out_specs=pl.BlockSpec((BN, 1, T, HID), lambda i: (i, 0, 0, 0)),
        out_shape=jax.ShapeDtypeStruct((N_DST, 1, T, HID), jnp.float32),
    )(aggs, W, b)


@jax.jit
def kernel(x, edge_src, edge_dst, W, b):
    esrc = edge_src.astype(jnp.int32)
    edst = edge_dst.astype(jnp.int32)
    xp = jnp.pad(x.reshape(T, N_SRC), ((0, 0), (0, N_SRC_P - N_SRC)))
    xp = xp.reshape(T, NS, XW)
    aggs = _sc_graph_agg(esrc, edst, xp)
    return _tc_expand(aggs, W.astype(jnp.float32), b.astype(jnp.float32))
